# Initial kernel scaffold; baseline (speedup 1.0000x reference)
import jax
import jax.numpy as jnp
from jax.experimental import pallas as pl


def kernel(input, index, weight, W, bn_gamma, bn_beta):
    raise NotImplementedError("write your pallas kernel here")


# speedup vs baseline: 4.2016x; 4.2016x over previous
import functools

import jax
import jax.numpy as jnp
from jax.experimental import pallas as pl

K = 16
_EPS = 1e-5


def _stats_kernel(x_ref, idx_ref, w_ref, sums_ref, cnt_ref, sumsq_ref):
    x = x_ref[0]
    w = w_ref[...]
    x2 = jax.lax.dot_general(w, x, (((0,), (0,)), ((), ())),
                             preferred_element_type=jnp.float32)
    idx = idx_ref[0, 0]
    hw = x.shape[1]
    oh = (idx[None, :] == jax.lax.broadcasted_iota(jnp.int32, (K, hw), 0)
          ).astype(jnp.float32)
    sums = jax.lax.dot_general(oh, x2, (((1,), (1,)), ((), ())),
                               preferred_element_type=jnp.float32)
    cnt = jnp.sum(oh, axis=1)
    sumsq = jnp.sum(x2 * x2, axis=1)
    sums_ref[0] = sums
    cnt_ref[0] = jnp.broadcast_to(cnt[:, None], cnt_ref.shape[1:])
    sumsq_ref[0] = sumsq[None, :]


def _adj_kernel(sums_ref, cnt_ref, sumsq_ref, wm_ref, g_ref, b_ref,
                a_ref, scale_ref):
    sums = sums_ref[...]
    cnt = cnt_ref[...][:, :, 0]
    sumsq = sumsq_ref[...][:, 0]
    wm = wm_ref[...]
    bsz, k, o = sums.shape
    denom = cnt + (cnt == 0).astype(jnp.float32)
    means = sums / denom[:, :, None]
    m = jax.lax.dot_general(wm, wm, (((1,), (1,)), ((), ())),
                            preferred_element_type=jnp.float32)
    d = means[:, None, :, :] - means[:, :, None, :]
    dr = d.reshape(bsz * k * k, o)
    dm = jax.lax.dot_general(dr, m, (((1,), (0,)), ((), ())),
                             preferred_element_type=jnp.float32)
    q = jnp.sum(dm * dr, axis=1).reshape(bsz, k, k)
    ii = jax.lax.broadcasted_iota(jnp.int32, (k, k), 0)
    jj = jax.lax.broadcasted_iota(jnp.int32, (k, k), 1)
    offdiag = (ii != jj).astype(jnp.float32)
    adjn = jnp.exp(-q) * offdiag[None]
    adjm = jnp.stack([
        jax.lax.dot_general(adjn[b], means[b], (((1,), (0,)), ((), ())),
                            preferred_element_type=jnp.float32)
        for b in range(bsz)])
    n = jnp.sum(cnt)
    tot = jnp.sum(sums, axis=(0, 1)) + jnp.sum(cnt[:, :, None] * adjm,
                                               axis=(0, 1))
    totsq = (jnp.sum(sumsq, axis=0)
             + 2.0 * jnp.sum(adjm * sums, axis=(0, 1))
             + jnp.sum(cnt[:, :, None] * adjm * adjm, axis=(0, 1)))
    mu = tot / n
    var = totsq / n - mu * mu
    scale = g_ref[0] * jax.lax.rsqrt(var + _EPS)
    shift = b_ref[0] - mu * scale
    a_ref[...] = adjm * scale[None, None, :] + shift[None, None, :]
    scale_ref[...] = scale[None, :]


def _apply_kernel(x_ref, idx_ref, w_ref, a_ref, scale_ref, out_ref):
    x = x_ref[0]
    w = w_ref[...]
    x2 = jax.lax.dot_general(w, x, (((0,), (0,)), ((), ())),
                             preferred_element_type=jnp.float32)
    idx = idx_ref[0, 0]
    hw = x.shape[1]
    oh = (idx[None, :] == jax.lax.broadcasted_iota(jnp.int32, (K, hw), 0)
          ).astype(jnp.float32)
    a = a_ref[0]
    g = jax.lax.dot_general(a, oh, (((0,), (0,)), ((), ())),
                            preferred_element_type=jnp.float32)
    s = scale_ref[0]
    out_ref[0] = s[:, None] * x2 + g


@functools.partial(jax.jit, static_argnames=())
def kernel(input, index, weight, W, bn_gamma, bn_beta):
    bsz, c, h, wsp = input.shape
    o = weight.shape[1]
    hw = h * wsp
    f32 = jnp.float32

    ih, iw = index.shape[2], index.shape[3]
    if (ih, iw) != (h, wsp):
        rows = (jnp.arange(h) * ih) // h
        cols = (jnp.arange(wsp) * iw) // wsp
        index = index[:, :, rows[:, None], cols[None, :]]
    idx3 = (index.reshape(bsz, 1, hw) - 1).astype(jnp.int32)
    xr = input.reshape(bsz, c, hw)

    sums, cnt, sumsq = pl.pallas_call(
        _stats_kernel,
        grid=(bsz,),
        in_specs=[
            pl.BlockSpec((1, c, hw), lambda b: (b, 0, 0)),
            pl.BlockSpec((1, 1, hw), lambda b: (b, 0, 0)),
            pl.BlockSpec((c, o), lambda b: (0, 0)),
        ],
        out_specs=[
            pl.BlockSpec((1, K, o), lambda b: (b, 0, 0)),
            pl.BlockSpec((1, K, o), lambda b: (b, 0, 0)),
            pl.BlockSpec((1, 1, o), lambda b: (b, 0, 0)),
        ],
        out_shape=[
            jax.ShapeDtypeStruct((bsz, K, o), f32),
            jax.ShapeDtypeStruct((bsz, K, o), f32),
            jax.ShapeDtypeStruct((bsz, 1, o), f32),
        ],
    )(xr, idx3, weight)

    a_tab, scale = pl.pallas_call(
        _adj_kernel,
        grid=(1,),
        in_specs=[
            pl.BlockSpec((bsz, K, o), lambda i: (0, 0, 0)),
            pl.BlockSpec((bsz, K, o), lambda i: (0, 0, 0)),
            pl.BlockSpec((bsz, 1, o), lambda i: (0, 0, 0)),
            pl.BlockSpec((o, o), lambda i: (0, 0)),
            pl.BlockSpec((1, o), lambda i: (0, 0)),
            pl.BlockSpec((1, o), lambda i: (0, 0)),
        ],
        out_specs=[
            pl.BlockSpec((bsz, K, o), lambda i: (0, 0, 0)),
            pl.BlockSpec((1, o), lambda i: (0, 0)),
        ],
        out_shape=[
            jax.ShapeDtypeStruct((bsz, K, o), f32),
            jax.ShapeDtypeStruct((1, o), f32),
        ],
    )(sums, cnt, sumsq, W, bn_gamma.reshape(1, o), bn_beta.reshape(1, o))

    out = pl.pallas_call(
        _apply_kernel,
        grid=(bsz,),
        in_specs=[
            pl.BlockSpec((1, c, hw), lambda b: (b, 0, 0)),
            pl.BlockSpec((1, 1, hw), lambda b: (b, 0, 0)),
            pl.BlockSpec((c, o), lambda b: (0, 0)),
            pl.BlockSpec((1, K, o), lambda b: (b, 0, 0)),
            pl.BlockSpec((1, o), lambda b: (0, 0)),
        ],
        out_specs=pl.BlockSpec((1, o, hw), lambda b: (b, 0, 0)),
        out_shape=jax.ShapeDtypeStruct((bsz, o, hw), f32),
    )(xr, idx3, weight, a_tab, scale)

    return out.reshape(bsz, o, h, wsp)


# speedup vs baseline: 5.0065x; 1.1916x over previous
import jax
import jax.numpy as jnp
from jax.experimental import pallas as pl
from jax.experimental.pallas import tpu as pltpu

K = 16
_EPS = 1e-5


def _make_fused(bsz, c, o, hw):
    def fused(x_ref, idx_ref, w_ref, wm_ref, g_ref, b_ref, out_ref,
              x2s, sums_s, cnt_s, sumsq_s, a_s, scale_s):
        i = pl.program_id(0)

        @pl.when(i < bsz)
        def _stats():
            x = x_ref[0]
            w = w_ref[...]
            x2 = jax.lax.dot_general(w, x, (((0,), (0,)), ((), ())),
                                     preferred_element_type=jnp.float32)
            x2s[pl.ds(i, 1)] = x2[None]
            idx = idx_ref[0, 0]
            oh = (idx[None, :] ==
                  jax.lax.broadcasted_iota(jnp.int32, (K, hw), 0)
                  ).astype(jnp.float32)
            sums = jax.lax.dot_general(oh, x2, (((1,), (1,)), ((), ())),
                                       preferred_element_type=jnp.float32)
            sums_s[pl.ds(i, 1)] = sums[None]
            cnt_s[pl.ds(i, 1)] = jnp.sum(oh, axis=1)[None]
            sumsq_s[pl.ds(i, 1)] = jnp.sum(x2 * x2, axis=1)[None]

        @pl.when(i == bsz)
        def _graph():
            sums = sums_s[...]
            cnt = cnt_s[...]
            sumsq = sumsq_s[...]
            wm = wm_ref[...]
            denom = cnt + (cnt == 0).astype(jnp.float32)
            means = sums / denom[:, :, None]
            m = jax.lax.dot_general(wm, wm, (((1,), (1,)), ((), ())),
                                    preferred_element_type=jnp.float32)
            d = means[:, None, :, :] - means[:, :, None, :]
            dr = d.reshape(bsz * K * K, o)
            dm = jax.lax.dot_general(dr, m, (((1,), (0,)), ((), ())),
                                     preferred_element_type=jnp.float32)
            q = jnp.sum(dm * dr, axis=1).reshape(bsz, K, K)
            ii = jax.lax.broadcasted_iota(jnp.int32, (K, K), 0)
            jj = jax.lax.broadcasted_iota(jnp.int32, (K, K), 1)
            offdiag = (ii != jj).astype(jnp.float32)
            adjn = jnp.exp(-q) * offdiag[None]
            adjm = jnp.stack([
                jax.lax.dot_general(adjn[b], means[b],
                                    (((1,), (0,)), ((), ())),
                                    preferred_element_type=jnp.float32)
                for b in range(bsz)])
            n = jnp.sum(cnt)
            tot = (jnp.sum(sums, axis=(0, 1))
                   + jnp.sum(cnt[:, :, None] * adjm, axis=(0, 1)))
            totsq = (jnp.sum(sumsq, axis=0)
                     + 2.0 * jnp.sum(adjm * sums, axis=(0, 1))
                     + jnp.sum(cnt[:, :, None] * adjm * adjm, axis=(0, 1)))
            mu = tot / n
            var = totsq / n - mu * mu
            scale = g_ref[0] * jax.lax.rsqrt(var + _EPS)
            shift = b_ref[0] - mu * scale
            a_s[...] = adjm * scale[None, None, :] + shift[None, None, :]
            scale_s[...] = scale[None, :]

        @pl.when(i >= bsz)
        def _apply():
            b = i - bsz
            x2 = x2s[b]
            idx = idx_ref[0, 0]
            oh = (idx[None, :] ==
                  jax.lax.broadcasted_iota(jnp.int32, (K, hw), 0)
                  ).astype(jnp.float32)
            a = a_s[b]
            g = jax.lax.dot_general(a, oh, (((0,), (0,)), ((), ())),
                                    preferred_element_type=jnp.float32)
            out_ref[0] = scale_s[0][:, None] * x2 + g

    return fused


def kernel(input, index, weight, W, bn_gamma, bn_beta):
    bsz, c, h, wsp = input.shape
    o = weight.shape[1]
    hw = h * wsp
    f32 = jnp.float32

    ih, iw = index.shape[2], index.shape[3]
    if (ih, iw) != (h, wsp):
        rows = (jnp.arange(h) * ih) // h
        cols = (jnp.arange(wsp) * iw) // wsp
        index = index[:, :, rows[:, None], cols[None, :]]
    idx3 = (index.reshape(bsz, 1, hw) - 1).astype(jnp.int32)
    xr = input.reshape(bsz, c, hw)

    out = pl.pallas_call(
        _make_fused(bsz, c, o, hw),
        grid=(2 * bsz,),
        in_specs=[
            pl.BlockSpec((1, c, hw), lambda i: (jnp.minimum(i, bsz - 1), 0, 0)),
            pl.BlockSpec((1, 1, hw),
                         lambda i: (jnp.where(i < bsz, i, i - bsz), 0, 0)),
            pl.BlockSpec((c, o), lambda i: (0, 0)),
            pl.BlockSpec((o, o), lambda i: (0, 0)),
            pl.BlockSpec((1, o), lambda i: (0, 0)),
            pl.BlockSpec((1, o), lambda i: (0, 0)),
        ],
        out_specs=pl.BlockSpec((1, o, hw),
                               lambda i: (jnp.maximum(i - bsz, 0), 0, 0)),
        out_shape=jax.ShapeDtypeStruct((bsz, o, hw), f32),
        scratch_shapes=[
            pltpu.VMEM((bsz, o, hw), f32),
            pltpu.VMEM((bsz, K, o), f32),
            pltpu.VMEM((bsz, K), f32),
            pltpu.VMEM((bsz, o), f32),
            pltpu.VMEM((bsz, K, o), f32),
            pltpu.VMEM((1, o), f32),
        ],
    )(xr, idx3, weight, W, bn_gamma.reshape(1, o), bn_beta.reshape(1, o))

    return out.reshape(bsz, o, h, wsp)


# speedup vs baseline: 5.0129x; 1.0013x over previous
import jax
import jax.numpy as jnp
from jax.experimental import pallas as pl
from jax.experimental.pallas import tpu as pltpu

K = 16
_EPS = 1e-5


def _make_fused(bsz, c, o, hw):
    def fused(x_ref, idx_ref, w_ref, wm_ref, g_ref, b_ref, out_ref,
              x2s, sums_s, cnt_s, sumsq_s, a_s, scale_s):
        i = pl.program_id(0)

        @pl.when(i < bsz)
        def _stats():
            x = x_ref[0].astype(jnp.bfloat16)
            w = w_ref[...].astype(jnp.bfloat16)
            x2 = jax.lax.dot_general(w, x, (((0,), (0,)), ((), ())),
                                     preferred_element_type=jnp.float32)
            x2s[pl.ds(i, 1)] = x2[None]
            idx = idx_ref[0, 0]
            oh = (idx[None, :] ==
                  jax.lax.broadcasted_iota(jnp.int32, (K, hw), 0)
                  ).astype(jnp.bfloat16)
            sums = jax.lax.dot_general(oh, x2.astype(jnp.bfloat16),
                                       (((1,), (1,)), ((), ())),
                                       preferred_element_type=jnp.float32)
            sums_s[pl.ds(i, 1)] = sums[None]
            cnt_s[pl.ds(i, 1)] = jnp.sum(oh.astype(jnp.float32), axis=1)[None]
            sumsq_s[pl.ds(i, 1)] = jnp.sum(x2 * x2, axis=1)[None]

        @pl.when(i == bsz)
        def _graph():
            sums = sums_s[...]
            cnt = cnt_s[...]
            sumsq = sumsq_s[...]
            wm = wm_ref[...]
            denom = cnt + (cnt == 0).astype(jnp.float32)
            means = sums / denom[:, :, None]
            m = jax.lax.dot_general(wm, wm, (((1,), (1,)), ((), ())),
                                    preferred_element_type=jnp.float32)
            d = means[:, None, :, :] - means[:, :, None, :]
            dr = d.reshape(bsz * K * K, o)
            dm = jax.lax.dot_general(dr, m, (((1,), (0,)), ((), ())),
                                     preferred_element_type=jnp.float32)
            q = jnp.sum(dm * dr, axis=1).reshape(bsz, K, K)
            ii = jax.lax.broadcasted_iota(jnp.int32, (K, K), 0)
            jj = jax.lax.broadcasted_iota(jnp.int32, (K, K), 1)
            offdiag = (ii != jj).astype(jnp.float32)
            adjn = jnp.exp(-q) * offdiag[None]
            adjm = jnp.stack([
                jax.lax.dot_general(adjn[b], means[b],
                                    (((1,), (0,)), ((), ())),
                                    preferred_element_type=jnp.float32)
                for b in range(bsz)])
            n = jnp.sum(cnt)
            tot = (jnp.sum(sums, axis=(0, 1))
                   + jnp.sum(cnt[:, :, None] * adjm, axis=(0, 1)))
            totsq = (jnp.sum(sumsq, axis=0)
                     + 2.0 * jnp.sum(adjm * sums, axis=(0, 1))
                     + jnp.sum(cnt[:, :, None] * adjm * adjm, axis=(0, 1)))
            mu = tot / n
            var = totsq / n - mu * mu
            scale = g_ref[0] * jax.lax.rsqrt(var + _EPS)
            shift = b_ref[0] - mu * scale
            a_s[...] = adjm * scale[None, None, :] + shift[None, None, :]
            scale_s[...] = scale[None, :]

        @pl.when(i >= bsz)
        def _apply():
            b = i - bsz
            x2 = x2s[b]
            idx = idx_ref[0, 0]
            oh = (idx[None, :] ==
                  jax.lax.broadcasted_iota(jnp.int32, (K, hw), 0)
                  ).astype(jnp.float32)
            a = a_s[b]
            g = jax.lax.dot_general(a, oh, (((0,), (0,)), ((), ())),
                                    preferred_element_type=jnp.float32)
            out_ref[0] = scale_s[0][:, None] * x2 + g

    return fused


def kernel(input, index, weight, W, bn_gamma, bn_beta):
    bsz, c, h, wsp = input.shape
    o = weight.shape[1]
    hw = h * wsp
    f32 = jnp.float32

    ih, iw = index.shape[2], index.shape[3]
    if (ih, iw) != (h, wsp):
        rows = (jnp.arange(h) * ih) // h
        cols = (jnp.arange(wsp) * iw) // wsp
        index = index[:, :, rows[:, None], cols[None, :]]
    idx3 = (index.reshape(bsz, 1, hw) - 1).astype(jnp.int32)
    xr = input.reshape(bsz, c, hw)

    out = pl.pallas_call(
        _make_fused(bsz, c, o, hw),
        grid=(2 * bsz,),
        in_specs=[
            pl.BlockSpec((1, c, hw), lambda i: (jnp.minimum(i, bsz - 1), 0, 0)),
            pl.BlockSpec((1, 1, hw),
                         lambda i: (jnp.where(i < bsz, i, i - bsz), 0, 0)),
            pl.BlockSpec((c, o), lambda i: (0, 0)),
            pl.BlockSpec((o, o), lambda i: (0, 0)),
            pl.BlockSpec((1, o), lambda i: (0, 0)),
            pl.BlockSpec((1, o), lambda i: (0, 0)),
        ],
        out_specs=pl.BlockSpec((1, o, hw),
                               lambda i: (jnp.maximum(i - bsz, 0), 0, 0)),
        out_shape=jax.ShapeDtypeStruct((bsz, o, hw), f32),
        scratch_shapes=[
            pltpu.VMEM((bsz, o, hw), f32),
            pltpu.VMEM((bsz, K, o), f32),
            pltpu.VMEM((bsz, K), f32),
            pltpu.VMEM((bsz, o), f32),
            pltpu.VMEM((bsz, K, o), f32),
            pltpu.VMEM((1, o), f32),
        ],
    )(xr, idx3, weight, W, bn_gamma.reshape(1, o), bn_beta.reshape(1, o))

    return out.reshape(bsz, o, h, wsp)
